# Initial kernel scaffold; baseline (speedup 1.0000x reference)
import jax
import jax.numpy as jnp
from jax.experimental import pallas as pl


def kernel(edge_index, x_s, x_t, W0, W1):
    raise NotImplementedError("write your pallas kernel here")


# speedup vs baseline: 20.8329x; 20.8329x over previous
import functools

import jax
import jax.numpy as jnp
from jax import lax
from jax.experimental import pallas as pl
from jax.experimental.pallas import tpu as pltpu
from jax.experimental.pallas import tpu_sc as plsc

NC = 2
NSUB = 16
NW = NC * NSUB
CH = 128
LANES = 16


def _mm0_body(xs_ref, xt_ref, w_ref, hs_ref, ht_ref):
    w = w_ref[...]
    hs_ref[...] = lax.dot_general(
        xs_ref[...], w, (((1,), (0,)), ((), ())),
        precision=lax.Precision.HIGHEST, preferred_element_type=jnp.float32)
    ht_ref[...] = lax.dot_general(
        xt_ref[...], w, (((1,), (0,)), ((), ())),
        precision=lax.Precision.HIGHEST, preferred_element_type=jnp.float32)


def _fin_body(hs_ref, ht_ref, a0_ref, a1_ref, d0_ref, d1_ref, w1_ref,
              ys_ref, yt_ref):
    w1 = w1_ref[...]
    ys_ref[...] = lax.dot_general(
        jnp.maximum(hs_ref[...], 0.0), w1, (((1,), (0,)), ((), ())),
        precision=lax.Precision.HIGHEST, preferred_element_type=jnp.float32)
    indeg = d0_ref[:, 0:1] + d1_ref[:, 0:1]
    dinv = lax.rsqrt(indeg + 1.0)
    z = dinv * dinv * ht_ref[...] + dinv * (a0_ref[...] + a1_ref[...])
    yt_ref[...] = lax.dot_general(
        jnp.maximum(z, 0.0), w1, (((1,), (0,)), ((), ())),
        precision=lax.Precision.HIGHEST, preferred_element_type=jnp.float32)


def _make_sc_aggregate(n_src, n_acc, cpt, d):
    rows_per_tile = n_acc // NSUB
    mesh = plsc.VectorSubcoreMesh(core_axis_name="c", subcore_axis_name="s")

    @functools.partial(
        pl.kernel,
        out_type=(
            jax.ShapeDtypeStruct((NC, n_acc, d), jnp.float32),
            jax.ShapeDtypeStruct((NC, n_acc, LANES), jnp.float32),
        ),
        mesh=mesh,
        scratch_types=[
            pltpu.VMEM_SHARED((n_acc, d), jnp.float32),
            pltpu.VMEM_SHARED((n_acc, LANES), jnp.float32),
            pltpu.VMEM((cpt, CH), jnp.int32),
            pltpu.VMEM((cpt, CH), jnp.int32),
            pltpu.VMEM((CH, d), jnp.float32),
            pltpu.VMEM((CH, d), jnp.float32),
            pltpu.VMEM((CH, LANES), jnp.float32),
            pltpu.SemaphoreType.DMA,
            pltpu.SemaphoreType.DMA,
        ],
    )
    def sc_agg(hs_hbm, rowi_hbm, coli_hbm, acc_out, deg_out,
               acc_sh, deg_sh, row_v, col_v, buf_a, buf_b, ones_v,
               sem_a, sem_b):
        cid = lax.axis_index("c")
        sid = lax.axis_index("s")
        wid = cid * NSUB + sid

        @pl.loop(0, CH)
        def _(i):
            @pl.loop(0, d, step=LANES)
            def _(j):
                buf_a[i, pl.ds(j, LANES)] = jnp.zeros((LANES,), jnp.float32)

        @pl.loop(0, CH)
        def _(i):
            ones_v[i, pl.ds(0, LANES)] = jnp.zeros((LANES,), jnp.float32)

        base = sid * rows_per_tile
        n_full = rows_per_tile // CH
        rem = rows_per_tile - n_full * CH

        @pl.loop(0, n_full)
        def _(k):
            pltpu.sync_copy(buf_a, acc_sh.at[pl.ds(base + k * CH, CH)])
            pltpu.sync_copy(ones_v, deg_sh.at[pl.ds(base + k * CH, CH)])

        if rem:
            pltpu.sync_copy(buf_a.at[pl.ds(0, rem)],
                            acc_sh.at[pl.ds(base + n_full * CH, rem)])
            pltpu.sync_copy(ones_v.at[pl.ds(0, rem)],
                            deg_sh.at[pl.ds(base + n_full * CH, rem)])

        @pl.loop(0, CH)
        def _(i):
            ones_v[i, pl.ds(0, LANES)] = jnp.ones((LANES,), jnp.float32)

        pltpu.sync_copy(rowi_hbm.at[wid], row_v)
        pltpu.sync_copy(coli_hbm.at[wid], col_v)

        plsc.subcore_barrier()

        pltpu.async_copy(hs_hbm.at[row_v.at[0]], buf_a, sem_a)

        @pl.loop(0, cpt, step=2)
        def _(j):
            pltpu.make_async_copy(hs_hbm.at[row_v.at[j]], buf_a, sem_a).wait()
            pltpu.async_copy(hs_hbm.at[row_v.at[j + 1]], buf_b, sem_b)
            pltpu.sync_copy(buf_a, acc_sh.at[col_v.at[j]], add=True)
            pltpu.sync_copy(ones_v, deg_sh.at[col_v.at[j]], add=True)
            pltpu.make_async_copy(hs_hbm.at[row_v.at[j + 1]], buf_b,
                                  sem_b).wait()

            @pl.when(j + 2 < cpt)
            def _():
                pltpu.async_copy(hs_hbm.at[row_v.at[j + 2]], buf_a, sem_a)

            pltpu.sync_copy(buf_b, acc_sh.at[col_v.at[j + 1]], add=True)
            pltpu.sync_copy(ones_v, deg_sh.at[col_v.at[j + 1]], add=True)

        plsc.subcore_barrier()

        pltpu.sync_copy(acc_sh.at[pl.ds(base, rows_per_tile)],
                        acc_out.at[cid].at[pl.ds(base, rows_per_tile)])
        pltpu.sync_copy(deg_sh.at[pl.ds(base, rows_per_tile)],
                        deg_out.at[cid].at[pl.ds(base, rows_per_tile)])

    return sc_agg


def _impl(edge_index, x_s, x_t, W0, W1):
    n_s, d_in = x_s.shape
    n_t = x_t.shape[0]
    d_hid = W0.shape[1]
    d_out = W1.shape[1]
    e = edge_index.shape[1]

    row = edge_index[0].astype(jnp.int32)
    col = edge_index[1].astype(jnp.int32)

    cpt = -(-e // (NW * CH))
    cpt = cpt + (cpt % 2)
    e_pad = NW * cpt * CH
    row_p = jnp.concatenate([row, jnp.zeros((e_pad - e,), jnp.int32)])
    col_p = jnp.concatenate([col, jnp.full((e_pad - e,), n_t, jnp.int32)])
    row_p = row_p.reshape(NW, cpt, CH)
    col_p = col_p.reshape(NW, cpt, CH)

    n_acc = -(-(n_t + 1) // (NSUB * 8)) * (NSUB * 8)

    hs, ht = pl.pallas_call(
        _mm0_body,
        out_shape=(
            jax.ShapeDtypeStruct((n_s, d_hid), jnp.float32),
            jax.ShapeDtypeStruct((n_t, d_hid), jnp.float32),
        ),
    )(x_s, x_t, W0)

    sc_agg = _make_sc_aggregate(n_s, n_acc, cpt, d_hid)
    acc, deg = sc_agg(hs, row_p, col_p)

    ys, yt = pl.pallas_call(
        _fin_body,
        out_shape=(
            jax.ShapeDtypeStruct((n_s, d_out), jnp.float32),
            jax.ShapeDtypeStruct((n_t, d_out), jnp.float32),
        ),
    )(hs, ht, acc[0, :n_t], acc[1, :n_t], deg[0, :n_t], deg[1, :n_t], W1)

    return ys, yt


_impl_jit = jax.jit(_impl)


def kernel(edge_index, x_s, x_t, W0, W1):
    return _impl_jit(edge_index, x_s, x_t, W0, W1)


# speedup vs baseline: 27.0420x; 1.2980x over previous
import functools

import jax
import jax.numpy as jnp
from jax import lax
from jax.experimental import pallas as pl
from jax.experimental.pallas import tpu as pltpu
from jax.experimental.pallas import tpu_sc as plsc

NC = 2
NSUB = 16
NW = NC * NSUB
CH = 128
LANES = 16
KB = 16


def _mm0_body(xs_ref, xt_ref, w_ref, hs_ref, ht_ref):
    w = w_ref[...]
    hs_ref[...] = lax.dot_general(
        xs_ref[...], w, (((1,), (0,)), ((), ())),
        precision=lax.Precision.HIGHEST, preferred_element_type=jnp.float32)
    ht_ref[...] = lax.dot_general(
        xt_ref[...], w, (((1,), (0,)), ((), ())),
        precision=lax.Precision.HIGHEST, preferred_element_type=jnp.float32)


def _fin_body(hs_ref, ht_ref, a0_ref, a1_ref, d0_ref, d1_ref, w1_ref,
              ys_ref, yt_ref):
    w1 = w1_ref[...]
    ys_ref[...] = lax.dot_general(
        jnp.maximum(hs_ref[...], 0.0), w1, (((1,), (0,)), ((), ())),
        precision=lax.Precision.HIGHEST, preferred_element_type=jnp.float32)
    indeg = d0_ref[:, 0:1] + d1_ref[:, 0:1]
    dinv = lax.rsqrt(indeg + 1.0)
    z = dinv * dinv * ht_ref[...] + dinv * (a0_ref[...] + a1_ref[...])
    yt_ref[...] = lax.dot_general(
        jnp.maximum(z, 0.0), w1, (((1,), (0,)), ((), ())),
        precision=lax.Precision.HIGHEST, preferred_element_type=jnp.float32)


def _make_sc_aggregate(n_src, n_acc, cpt, d):
    rows_per_tile = n_acc // NSUB
    src_per_tile = n_src // NSUB
    mesh = plsc.VectorSubcoreMesh(core_axis_name="c", subcore_axis_name="s")

    @functools.partial(
        pl.kernel,
        out_type=(
            jax.ShapeDtypeStruct((NC, n_acc, d), jnp.float32),
            jax.ShapeDtypeStruct((NC, n_acc, LANES), jnp.float32),
        ),
        mesh=mesh,
        scratch_types=[
            pltpu.VMEM_SHARED((n_acc, d), jnp.float32),
            pltpu.VMEM_SHARED((n_acc, LANES), jnp.float32),
            pltpu.VMEM((KB, CH), jnp.int32),
            pltpu.VMEM((KB, CH), jnp.int32),
            pltpu.VMEM((CH, d), jnp.float32),
            pltpu.VMEM((CH, d), jnp.float32),
            pltpu.VMEM((CH, LANES), jnp.float32),
            pltpu.SemaphoreType.DMA,
            pltpu.SemaphoreType.DMA,
        ],
    )
    def sc_agg(hs_hbm, rowi_hbm, coli_hbm, acc_out, deg_out,
               acc_sh, deg_sh, row_v, col_v, buf_a, buf_b, ones_v,
               sem_a, sem_b):
        cid = lax.axis_index("c")
        sid = lax.axis_index("s")
        wid = cid * NSUB + sid

        @pl.loop(0, CH)
        def _(i):
            @pl.loop(0, d, step=LANES)
            def _(j):
                buf_a[i, pl.ds(j, LANES)] = jnp.zeros((LANES,), jnp.float32)

        @pl.loop(0, CH)
        def _(i):
            ones_v[i, pl.ds(0, LANES)] = jnp.zeros((LANES,), jnp.float32)

        base = sid * rows_per_tile
        n_full = rows_per_tile // CH
        rem = rows_per_tile - n_full * CH

        @pl.loop(0, n_full)
        def _(k):
            pltpu.sync_copy(buf_a, acc_sh.at[pl.ds(base + k * CH, CH)])
            pltpu.sync_copy(ones_v, deg_sh.at[pl.ds(base + k * CH, CH)])

        if rem:
            pltpu.sync_copy(buf_a.at[pl.ds(0, rem)],
                            acc_sh.at[pl.ds(base + n_full * CH, rem)])
            pltpu.sync_copy(ones_v.at[pl.ds(0, rem)],
                            deg_sh.at[pl.ds(base + n_full * CH, rem)])

        @pl.loop(0, CH)
        def _(i):
            ones_v[i, pl.ds(0, LANES)] = jnp.ones((LANES,), jnp.float32)

        plsc.subcore_barrier()

        @pl.loop(0, cpt // KB)
        def _(b):
            pltpu.sync_copy(rowi_hbm.at[wid].at[pl.ds(b * KB, KB)], row_v)
            pltpu.sync_copy(coli_hbm.at[wid].at[pl.ds(b * KB, KB)], col_v)
            pltpu.async_copy(hs_hbm.at[row_v.at[0]], buf_a, sem_a)

            @pl.loop(0, KB, step=2)
            def _(j):
                pltpu.make_async_copy(hs_hbm.at[row_v.at[j]], buf_a,
                                      sem_a).wait()
                pltpu.async_copy(hs_hbm.at[row_v.at[j + 1]], buf_b, sem_b)
                pltpu.sync_copy(buf_a, acc_sh.at[col_v.at[j]], add=True)
                pltpu.sync_copy(ones_v, deg_sh.at[col_v.at[j]], add=True)
                pltpu.make_async_copy(hs_hbm.at[row_v.at[j + 1]], buf_b,
                                      sem_b).wait()

                @pl.when(j + 2 < KB)
                def _():
                    pltpu.async_copy(hs_hbm.at[row_v.at[j + 2]], buf_a, sem_a)

                pltpu.sync_copy(buf_b, acc_sh.at[col_v.at[j + 1]], add=True)
                pltpu.sync_copy(ones_v, deg_sh.at[col_v.at[j + 1]], add=True)

        plsc.subcore_barrier()

        pltpu.sync_copy(acc_sh.at[pl.ds(base, rows_per_tile)],
                        acc_out.at[cid].at[pl.ds(base, rows_per_tile)])
        pltpu.sync_copy(deg_sh.at[pl.ds(base, rows_per_tile)],
                        deg_out.at[cid].at[pl.ds(base, rows_per_tile)])

    return sc_agg


def _impl(edge_index, x_s, x_t, W0, W1):
    n_s, d_in = x_s.shape
    n_t = x_t.shape[0]
    d_hid = W0.shape[1]
    d_out = W1.shape[1]
    e = edge_index.shape[1]

    row = edge_index[0].astype(jnp.int32)
    col = edge_index[1].astype(jnp.int32)

    cpt = -(-e // (NW * CH))
    cpt = -(-cpt // KB) * KB
    e_pad = NW * cpt * CH
    row_p = jnp.concatenate([row, jnp.zeros((e_pad - e,), jnp.int32)])
    col_p = jnp.concatenate([col, jnp.full((e_pad - e,), n_t, jnp.int32)])
    row_p = row_p.reshape(NW, cpt, CH)
    col_p = col_p.reshape(NW, cpt, CH)

    n_acc = -(-(n_t + 1) // (NSUB * 8)) * (NSUB * 8)
    n_sp = -(-n_s // (NSUB * 8)) * (NSUB * 8)
    xs_p = jnp.pad(x_s, ((0, n_sp - n_s), (0, 0)))

    hs_p, ht = pl.pallas_call(
        _mm0_body,
        out_shape=(
            jax.ShapeDtypeStruct((n_sp, d_hid), jnp.float32),
            jax.ShapeDtypeStruct((n_t, d_hid), jnp.float32),
        ),
    )(xs_p, x_t, W0)
    hs = hs_p[:n_s]

    sc_agg = _make_sc_aggregate(n_sp, n_acc, cpt, d_hid)
    acc, deg = sc_agg(hs_p, row_p, col_p)

    ys, yt = pl.pallas_call(
        _fin_body,
        out_shape=(
            jax.ShapeDtypeStruct((n_s, d_out), jnp.float32),
            jax.ShapeDtypeStruct((n_t, d_out), jnp.float32),
        ),
    )(hs, ht, acc[0, :n_t], acc[1, :n_t], deg[0, :n_t], deg[1, :n_t], W1)

    return ys, yt


_impl_jit = jax.jit(_impl)


def kernel(edge_index, x_s, x_t, W0, W1):
    return _impl_jit(edge_index, x_s, x_t, W0, W1)
